# Initial kernel scaffold; baseline (speedup 1.0000x reference)
import jax
import jax.numpy as jnp
from jax.experimental import pallas as pl


def kernel(x, edge_index, W1, b1, W2, b2):
    raise NotImplementedError("write your pallas kernel here")


# speedup vs baseline: 9.9643x; 9.9643x over previous
import functools

import jax
import jax.numpy as jnp
from jax import lax
from jax.experimental import pallas as pl
from jax.experimental.pallas import tpu as pltpu
from jax.experimental.pallas import tpu_sc as plsc

N = 10000
E = 320000
NP = 10240
NTILES = 32
NCHUNK = 80
EPT = NCHUNK * 128
ROWS_PT = NP // 16
DUMMY = N

_mesh = plsc.VectorSubcoreMesh(core_axis_name="c", subcore_axis_name="s")


def _deg_parts(dst2):
  @functools.partial(
      pl.kernel,
      out_type=jax.ShapeDtypeStruct((NTILES, NP), jnp.float32),
      mesh=_mesh,
      compiler_params=pltpu.CompilerParams(needs_layout_passes=False),
      scratch_types=[
          pltpu.VMEM((EPT,), jnp.int32),
          pltpu.VMEM((NP,), jnp.float32),
      ],
  )
  def k(dst_hbm, out_hbm, dstv, hist):
    c = lax.axis_index("c")
    s = lax.axis_index("s")
    wid = c * 16 + s
    pltpu.sync_copy(dst_hbm.at[wid], dstv)
    zeros = jnp.zeros((16,), jnp.float32)
    ones = jnp.ones((16,), jnp.float32)

    def zbody(i, carry):
      hist[pl.ds(i * 16, 16)] = zeros
      return carry

    lax.fori_loop(0, NP // 16, zbody, 0)

    def body(i, carry):
      idx = dstv[pl.ds(i * 16, 16)]
      plsc.addupdate_scatter(hist, [idx], ones)
      return carry

    lax.fori_loop(0, EPT // 16, body, 0)
    pltpu.sync_copy(hist, out_hbm.at[wid])

  return k(dst2)


def _edge_scatter(g, src3, dst3, d):
  @functools.partial(
      pl.kernel,
      out_type=jax.ShapeDtypeStruct((2, NP, d), jnp.float32),
      mesh=_mesh,
      scratch_types=[
          pltpu.VMEM((NCHUNK // 2, 128), jnp.int32),
          pltpu.VMEM((NCHUNK // 2, 128), jnp.int32),
          pltpu.VMEM((128, d), jnp.float32),
          pltpu.VMEM((128, d), jnp.float32),
          pltpu.VMEM_SHARED((NP, d), jnp.float32),
          pltpu.SemaphoreType.DMA,
          pltpu.SemaphoreType.DMA,
      ],
  )
  def k(g_hbm, src_hbm, dst_hbm, zz_hbm, out_hbm,
        srcv, dstv, bufA, bufB, acc, semA, semB):
    c = lax.axis_index("c")
    s = lax.axis_index("s")
    wid = c * 16 + s
    r0 = s * ROWS_PT
    pltpu.sync_copy(zz_hbm, acc.at[pl.ds(r0, ROWS_PT)])
    plsc.subcore_barrier()

    half = NCHUNK // 2
    for h in range(2):
      pltpu.sync_copy(src_hbm.at[wid, pl.ds(h * half, half)], srcv)
      pltpu.sync_copy(dst_hbm.at[wid, pl.ds(h * half, half)], dstv)

      pltpu.async_copy(g_hbm.at[srcv.at[0]], bufA, semA)
      pltpu.async_copy(g_hbm.at[srcv.at[1]], bufB, semB)

      def body(i, carry):
        jA = 2 * i
        jB = 2 * i + 1
        pltpu.make_async_copy(g_hbm.at[srcv.at[0]], bufA, semA).wait()
        pltpu.sync_copy(bufA, acc.at[dstv.at[jA]], add=True)
        nA = jnp.minimum(jA + 2, half - 2)
        pltpu.async_copy(g_hbm.at[srcv.at[nA]], bufA, semA)
        pltpu.make_async_copy(g_hbm.at[srcv.at[1]], bufB, semB).wait()
        pltpu.sync_copy(bufB, acc.at[dstv.at[jB]], add=True)
        nB = jnp.minimum(jB + 2, half - 1)
        pltpu.async_copy(g_hbm.at[srcv.at[nB]], bufB, semB)
        return carry

      lax.fori_loop(0, half // 2, body, 0)
      pltpu.make_async_copy(g_hbm.at[srcv.at[0]], bufA, semA).wait()
      pltpu.make_async_copy(g_hbm.at[srcv.at[1]], bufB, semB).wait()
    plsc.subcore_barrier()
    pltpu.sync_copy(acc.at[pl.ds(r0, ROWS_PT)],
                    out_hbm.at[c, pl.ds(r0, ROWS_PT)])

  return k(g, src3, dst3, jnp.zeros((ROWS_PT, d), jnp.float32))


def _dinv(deg_parts):
  def body(dp_ref, o_ref):
    deg = jnp.sum(dp_ref[...], axis=0) + 1.0
    o_ref[...] = lax.rsqrt(deg)[:, None]

  return pl.pallas_call(
      body,
      out_shape=jax.ShapeDtypeStruct((NP, 1), jnp.float32),
  )(deg_parts)


def _tc_scale_matmul(x, w, dinv):
  din, dout = w.shape

  def body(x_ref, w_ref, dv_ref, o_ref):
    h = jnp.dot(x_ref[...], w_ref[...], preferred_element_type=jnp.float32)
    o_ref[...] = h * dv_ref[...]

  return pl.pallas_call(
      body,
      grid=(10,),
      in_specs=[
          pl.BlockSpec((1000, din), lambda i: (i, 0)),
          pl.BlockSpec((din, dout), lambda i: (0, 0)),
          pl.BlockSpec((1000, 1), lambda i: (i, 0)),
      ],
      out_specs=pl.BlockSpec((1000, dout), lambda i: (i, 0)),
      out_shape=jax.ShapeDtypeStruct((N, dout), jnp.float32),
  )(x, w, dinv)


def _tc_combine_relu_matmul(sp, g, dinv, b, w):
  din, dout = w.shape

  def body(sp_ref, g_ref, dv_ref, b_ref, w_ref, o_ref):
    ssum = sp_ref[0] + sp_ref[1] + g_ref[...]
    a = ssum * dv_ref[...] + b_ref[...]
    r = jnp.maximum(a, 0.0)
    h = jnp.dot(r, w_ref[...], preferred_element_type=jnp.float32)
    o_ref[...] = h * dv_ref[...]

  return pl.pallas_call(
      body,
      grid=(10,),
      in_specs=[
          pl.BlockSpec((2, 1000, din), lambda i: (0, i, 0)),
          pl.BlockSpec((1000, din), lambda i: (i, 0)),
          pl.BlockSpec((1000, 1), lambda i: (i, 0)),
          pl.BlockSpec((1, din), lambda i: (0, 0)),
          pl.BlockSpec((din, dout), lambda i: (0, 0)),
      ],
      out_specs=pl.BlockSpec((1000, dout), lambda i: (i, 0)),
      out_shape=jax.ShapeDtypeStruct((N, dout), jnp.float32),
  )(sp, g, dinv, b, w)


def _tc_combine_logsoftmax(sp, g, dinv, b, dout):
  dpad = g.shape[1]

  def body(sp_ref, g_ref, dv_ref, b_ref, o_ref):
    full = (sp_ref[0] + sp_ref[1] + g_ref[...]) * dv_ref[...]
    o = full[:, :dout] + b_ref[...]
    m = jnp.max(o, axis=1, keepdims=True)
    e = jnp.exp(o - m)
    lse = jnp.log(jnp.sum(e, axis=1, keepdims=True))
    o_ref[...] = o - m - lse

  return pl.pallas_call(
      body,
      grid=(10,),
      in_specs=[
          pl.BlockSpec((2, 1000, dpad), lambda i: (0, i, 0)),
          pl.BlockSpec((1000, dpad), lambda i: (i, 0)),
          pl.BlockSpec((1000, 1), lambda i: (i, 0)),
          pl.BlockSpec((1, dout), lambda i: (0, 0)),
      ],
      out_specs=pl.BlockSpec((1000, dout), lambda i: (i, 0)),
      out_shape=jax.ShapeDtypeStruct((N, dout), jnp.float32),
  )(sp, g, dinv, b)


def kernel(x, edge_index, W1, b1, W2, b2):
  src = edge_index[0].astype(jnp.int32)
  dst = edge_index[1].astype(jnp.int32)
  pad = NTILES * EPT - E
  src_p = jnp.concatenate([src, jnp.zeros((pad,), jnp.int32)])
  dst_p = jnp.concatenate([dst, jnp.full((pad,), DUMMY, jnp.int32)])
  src3 = src_p.reshape(NTILES, NCHUNK, 128)
  dst3 = dst_p.reshape(NTILES, NCHUNK, 128)
  dst2 = dst_p.reshape(NTILES, EPT)

  dparts = _deg_parts(dst2)
  dinv = _dinv(dparts)

  g1 = _tc_scale_matmul(x, W1, dinv)
  s1 = _edge_scatter(g1, src3, dst3, W1.shape[1])
  W2p = jnp.pad(W2, ((0, 0), (0, 128 - W2.shape[1])))
  g2 = _tc_combine_relu_matmul(s1, g1, dinv, b1.reshape(1, -1), W2p)
  s2 = _edge_scatter(g2, src3, dst3, 128)
  return _tc_combine_logsoftmax(s2, g2, dinv, b2.reshape(1, -1), W2.shape[1])
